# Initial kernel scaffold; baseline (speedup 1.0000x reference)
import jax
import jax.numpy as jnp
from jax.experimental import pallas as pl


def kernel(tokens, table):
    raise NotImplementedError("write your pallas kernel here")


# speedup vs baseline: 2.4198x; 2.4198x over previous
import functools
import math

import jax
import jax.numpy as jnp
from jax import lax
from jax.experimental import pallas as pl
from jax.experimental.pallas import tpu as pltpu
from jax.experimental.pallas import tpu_sc as plsc

D = 128
SCALE = math.sqrt(float(D))

_info = plsc.get_sparse_core_info()
_NC = _info.num_cores
_NS = _info.num_subcores
_NW = _NC * _NS
_L = _info.num_lanes

B = 4096 * 50
B_PER_W = B // _NW
CH = 128
NCH = B_PER_W // CH

_mesh = plsc.VectorSubcoreMesh(core_axis_name="c", subcore_axis_name="s")


@functools.partial(
    pl.kernel,
    mesh=_mesh,
    out_type=jax.ShapeDtypeStruct((B, D), jnp.float32),
    scratch_types=[
        pltpu.VMEM((B_PER_W,), jnp.int32),
        pltpu.VMEM((CH, D), jnp.float32),
        pltpu.SemaphoreType.DMA,
    ],
)
def _gather_scale(idx_hbm, table_hbm, out_hbm, idx_v, buf, gsem):
    wid = lax.axis_index("s") * _NC + lax.axis_index("c")
    base = wid * B_PER_W
    pltpu.sync_copy(idx_hbm.at[pl.ds(base, B_PER_W)], idx_v)

    def chunk_body(cidx, carry):
        off = cidx * CH
        pltpu.async_copy(
            table_hbm.at[idx_v.at[pl.ds(off, CH)]], buf, gsem
        ).wait()

        def row_body(r, c2):
            for j in range(D // _L):
                sl = pl.ds(j * _L, _L)
                buf[r, sl] = buf[r, sl] * SCALE
            return c2

        lax.fori_loop(0, CH, row_body, 0)

        pltpu.sync_copy(buf, out_hbm.at[pl.ds(base + off, CH)])
        return carry

    lax.fori_loop(0, NCH, chunk_body, 0)


def kernel(tokens, table):
    idx = tokens.reshape(-1).astype(jnp.int32)
    out = _gather_scale(idx, table)
    return out.reshape(tokens.shape + (D,))


# speedup vs baseline: 2.8150x; 1.1633x over previous
import functools
import math

import jax
import jax.numpy as jnp
from jax import lax
from jax.experimental import pallas as pl
from jax.experimental.pallas import tpu as pltpu
from jax.experimental.pallas import tpu_sc as plsc

D = 128
SCALE = math.sqrt(float(D))

_info = plsc.get_sparse_core_info()
_NC = _info.num_cores
_NS = _info.num_subcores
_NW = _NC * _NS
_L = _info.num_lanes

B = 4096 * 50
B_PER_W = B // _NW
CH = 128
NCH = B_PER_W // CH
NBUF = 2

_mesh = plsc.VectorSubcoreMesh(core_axis_name="c", subcore_axis_name="s")


@functools.partial(
    pl.kernel,
    mesh=_mesh,
    out_type=jax.ShapeDtypeStruct((B, D), jnp.float32),
    scratch_types=[
        pltpu.VMEM((B_PER_W,), jnp.int32),
        pltpu.VMEM((NBUF, CH, D), jnp.float32),
        pltpu.SemaphoreType.DMA,
        pltpu.SemaphoreType.DMA,
    ],
)
def _gather_scale(idx_hbm, table_hbm, out_hbm, idx_v, bufs, gsem, osem):
    wid = lax.axis_index("s") * _NC + lax.axis_index("c")
    base = wid * B_PER_W
    pltpu.sync_copy(idx_hbm.at[pl.ds(base, B_PER_W)], idx_v)

    def gather(c, b):
        return pltpu.make_async_copy(
            table_hbm.at[idx_v.at[pl.ds(c * CH, CH)]], bufs.at[b], gsem
        )

    def out_copy(c, b):
        return pltpu.make_async_copy(
            bufs.at[b], out_hbm.at[pl.ds(base + c * CH, CH)], osem
        )

    def scale(b):
        def row_body(r, c2):
            for j in range(D // _L):
                sl = pl.ds(j * _L, _L)
                bufs[b, r, sl] = bufs[b, r, sl] * SCALE
            return c2

        lax.fori_loop(0, CH, row_body, 0, unroll=2)

    gather(0, 0).start()

    def chunk_body(c, carry):
        b = lax.rem(c, NBUF)
        bn = lax.rem(c + 1, NBUF)
        gather(c, b).wait()
        @pl.when(c + 1 < NCH)
        def _():
            @pl.when(c + 1 >= NBUF)
            def _():
                out_copy(c + 1 - NBUF, bn).wait()

            gather(c + 1, bn).start()

        scale(b)
        out_copy(c, b).start()
        return carry

    lax.fori_loop(0, NCH, chunk_body, 0)
    out_copy(NCH - NBUF, lax.rem(NCH - NBUF, NBUF)).wait()
    out_copy(NCH - 1, lax.rem(NCH - 1, NBUF)).wait()


def kernel(tokens, table):
    idx = tokens.reshape(-1).astype(jnp.int32)
    out = _gather_scale(idx, table)
    return out.reshape(tokens.shape + (D,))


# speedup vs baseline: 2.8169x; 1.0007x over previous
import functools
import math

import jax
import jax.numpy as jnp
from jax import lax
from jax.experimental import pallas as pl
from jax.experimental.pallas import tpu as pltpu
from jax.experimental.pallas import tpu_sc as plsc

D = 128
SCALE = math.sqrt(float(D))

_info = plsc.get_sparse_core_info()
_NC = _info.num_cores
_NS = _info.num_subcores
_NW = _NC * _NS
_L = _info.num_lanes

B = 4096 * 50
B_PER_W = B // _NW
CH = 128
NCH = B_PER_W // CH
NBUF = 2
NPAIR = NCH // NBUF

_mesh = plsc.VectorSubcoreMesh(core_axis_name="c", subcore_axis_name="s")


@functools.partial(
    pl.kernel,
    mesh=_mesh,
    out_type=jax.ShapeDtypeStruct((B, D), jnp.float32),
    scratch_types=[
        pltpu.VMEM((B_PER_W,), jnp.int32),
        pltpu.VMEM((CH, D), jnp.float32),
        pltpu.VMEM((CH, D), jnp.float32),
        pltpu.SemaphoreType.DMA,
        pltpu.SemaphoreType.DMA,
    ],
)
def _gather_scale(idx_hbm, table_hbm, out_hbm, idx_v, buf0, buf1, gsem, osem):
    bufs = (buf0, buf1)
    wid = lax.axis_index("s") * _NC + lax.axis_index("c")
    base = wid * B_PER_W
    pltpu.sync_copy(idx_hbm.at[pl.ds(base, B_PER_W)], idx_v)

    def gather(c, buf):
        return pltpu.make_async_copy(
            table_hbm.at[idx_v.at[pl.ds(c * CH, CH)]], buf, gsem
        )

    def out_copy(c, buf):
        return pltpu.make_async_copy(
            buf, out_hbm.at[pl.ds(base + c * CH, CH)], osem
        )

    def scale(buf):
        def row_body(r, c2):
            for j in range(D // _L):
                sl = pl.ds(j * _L, _L)
                buf[r, sl] = buf[r, sl] * SCALE
            return c2

        lax.fori_loop(0, CH, row_body, 0, unroll=2)

    gather(0, buf0).start()

    def pair_body(p, carry):
        for b in range(NBUF):
            c = p * NBUF + b
            buf = bufs[b]
            bufn = bufs[(b + 1) % NBUF]
            gather(c, buf).wait()

            @pl.when(c + 1 < NCH)
            def _():
                @pl.when(c + 1 >= NBUF)
                def _():
                    out_copy(c + 1 - NBUF, bufn).wait()

                gather(c + 1, bufn).start()

            scale(buf)
            out_copy(c, buf).start()
        return carry

    lax.fori_loop(0, NPAIR, pair_body, 0)
    out_copy(NCH - NBUF, bufs[(NCH - NBUF) % NBUF]).wait()
    out_copy(NCH - 1, bufs[(NCH - 1) % NBUF]).wait()


def kernel(tokens, table):
    idx = tokens.reshape(-1).astype(jnp.int32)
    out = _gather_scale(idx, table)
    return out.reshape(tokens.shape + (D,))


# speedup vs baseline: 5.1033x; 1.8117x over previous
import functools
import math

import jax
import jax.numpy as jnp
from jax import lax
from jax.experimental import pallas as pl
from jax.experimental.pallas import tpu as pltpu
from jax.experimental.pallas import tpu_sc as plsc

D = 128
T = 50
NSEQ = 4096
SCALE = math.sqrt(float(D))

_info = plsc.get_sparse_core_info()
_NC = _info.num_cores
_NS = _info.num_subcores
_NW = _NC * _NS
_L = _info.num_lanes

SEQ_PER_W = NSEQ // _NW
G = 4
CH = G * T
NCH = SEQ_PER_W // G
NBUF = 2

_mesh = plsc.VectorSubcoreMesh(core_axis_name="c", subcore_axis_name="s")


@functools.partial(
    pl.kernel,
    mesh=_mesh,
    out_type=jax.ShapeDtypeStruct((NSEQ, T, D), jnp.float32),
    scratch_types=[
        pltpu.VMEM((SEQ_PER_W * T,), jnp.int32),
        pltpu.VMEM((CH, D), jnp.float32),
        pltpu.VMEM((CH, D), jnp.float32),
        pltpu.SemaphoreType.DMA,
        pltpu.SemaphoreType.DMA,
    ],
    compiler_params=pltpu.CompilerParams(use_tc_tiling_on_sc=True),
)
def _gather_scale(idx_hbm, table_hbm, out_hbm, idx_v, buf0, buf1, gsem, osem):
    bufs = (buf0, buf1)
    wid = lax.axis_index("s") * _NC + lax.axis_index("c")
    base = wid * (SEQ_PER_W * T)
    seq_base = wid * SEQ_PER_W
    pltpu.sync_copy(idx_hbm.at[pl.ds(base, SEQ_PER_W * T)], idx_v)

    def gather(c, buf):
        return pltpu.make_async_copy(
            table_hbm.at[idx_v.at[pl.ds(c * CH, CH)]], buf, gsem
        )

    def out_copies(c, buf):
        return [
            pltpu.make_async_copy(
                buf.at[pl.ds(g * T, T)], out_hbm.at[seq_base + c * G + g], osem
            )
            for g in range(G)
        ]

    def out_start(c, buf):
        for cp in out_copies(c, buf):
            cp.start()

    def out_wait(c, buf):
        for cp in out_copies(c, buf):
            cp.wait()

    def scale(buf):
        def row_body(r, c2):
            for j in range(D // _L):
                sl = pl.ds(j * _L, _L)
                buf[r, sl] = buf[r, sl] * SCALE
            return c2

        lax.fori_loop(0, CH, row_body, 0, unroll=2)

    gather(0, buf0).start()

    def chunk_body(p, carry):
        for b in range(NBUF):
            c = p * NBUF + b
            buf = bufs[b]
            bufn = bufs[(b + 1) % NBUF]
            gather(c, buf).wait()

            @pl.when(c + 1 < NCH)
            def _():
                @pl.when(c + 1 >= NBUF)
                def _():
                    out_wait(c + 1 - NBUF, bufn)

                gather(c + 1, bufn).start()

            scale(buf)
            out_start(c, buf)
        return carry

    lax.fori_loop(0, NCH // NBUF, chunk_body, 0)
    out_wait(NCH - NBUF, bufs[(NCH - NBUF) % NBUF])
    out_wait(NCH - 1, bufs[(NCH - 1) % NBUF])


def kernel(tokens, table):
    idx = tokens.reshape(-1).astype(jnp.int32)
    return _gather_scale(idx, table)
